# baseline (speedup 1.0000x reference)
import jax
import jax.numpy as jnp
from jax.experimental import pallas as pl
from jax.experimental.pallas import tpu as pltpu

_TEMP = 0.1
_EPS = 1e-16
_BLK = 8000


def _mlp_body(ea_ref, W1_ref, b1_ref, W2_ref, b2_ref, pi_ref, bmax_ref):
    ea = ea_ref[...]
    h = jnp.maximum(
        jnp.dot(ea, W1_ref[...], preferred_element_type=jnp.float32) + b1_ref[...],
        0.0,
    )
    pi = jnp.dot(h, W2_ref[...], preferred_element_type=jnp.float32) + b2_ref[...]
    pi_ref[...] = pi
    bmax_ref[...] = jnp.max(pi).reshape(1, 1, 1)


def _exp_body(pi_ref, bmax_ref, e_ref, den_ref, macc_ref, sacc_ref):
    j = pl.program_id(0)

    @pl.when(j == 0)
    def _():
        macc_ref[0] = jnp.max(bmax_ref[...])
        sacc_ref[0] = 0.0

    ml = macc_ref[0] / _TEMP
    e = jnp.exp(pi_ref[...] / _TEMP - ml)
    e_ref[...] = e
    sacc_ref[0] += jnp.sum(e)

    @pl.when(j == pl.num_programs(0) - 1)
    def _():
        den_ref[...] = sacc_ref[0].reshape(1, 1)


def _score_body(e_ref, den_ref, score_ref):
    score_ref[...] = jnp.maximum(e_ref[...] / (den_ref[0, 0] + _EPS), 0.0)


def kernel(x, edge_index, edge_attr, batch, edge_batch, att, W1, b1, W2, b2):
    E = edge_attr.shape[0]
    grid = E // _BLK
    pi, bmax = pl.pallas_call(
        _mlp_body,
        grid=(grid,),
        in_specs=[
            pl.BlockSpec((_BLK, 2), lambda i: (i, 0)),
            pl.BlockSpec((2, 128), lambda i: (0, 0)),
            pl.BlockSpec((1, 128), lambda i: (0, 0)),
            pl.BlockSpec((128, 1), lambda i: (0, 0)),
            pl.BlockSpec((1, 1), lambda i: (0, 0)),
        ],
        out_specs=[
            pl.BlockSpec((_BLK, 1), lambda i: (i, 0)),
            pl.BlockSpec((1, 1, 1), lambda i: (i, 0, 0)),
        ],
        out_shape=[
            jax.ShapeDtypeStruct((E, 1), jnp.float32),
            jax.ShapeDtypeStruct((grid, 1, 1), jnp.float32),
        ],
    )(edge_attr, W1, b1.reshape(1, 128), W2, b2.reshape(1, 1))

    e, den = pl.pallas_call(
        _exp_body,
        grid=(grid,),
        in_specs=[
            pl.BlockSpec((_BLK, 1), lambda i: (i, 0)),
            pl.BlockSpec((grid, 1, 1), lambda i: (0, 0, 0)),
        ],
        out_specs=[
            pl.BlockSpec((_BLK, 1), lambda i: (i, 0)),
            pl.BlockSpec((1, 1), lambda i: (0, 0)),
        ],
        out_shape=[
            jax.ShapeDtypeStruct((E, 1), jnp.float32),
            jax.ShapeDtypeStruct((1, 1), jnp.float32),
        ],
        scratch_shapes=[
            pltpu.SMEM((1,), jnp.float32),
            pltpu.SMEM((1,), jnp.float32),
        ],
    )(pi, bmax)

    score = pl.pallas_call(
        _score_body,
        grid=(grid,),
        in_specs=[
            pl.BlockSpec((_BLK, 1), lambda i: (i, 0)),
            pl.BlockSpec((1, 1), lambda i: (0, 0)),
        ],
        out_specs=pl.BlockSpec((_BLK, 1), lambda i: (i, 0)),
        out_shape=jax.ShapeDtypeStruct((E, 1), jnp.float32),
    )(e, den)

    score = score.reshape(-1)
    k = E // 2
    _, perm = jax.lax.top_k(score, k)
    ei = edge_index[:, perm]
    ea2 = edge_attr[perm]
    used = jnp.zeros((x.shape[0],), dtype=bool).at[ei.reshape(-1)].set(True)
    new_idx = jnp.cumsum(used.astype(jnp.int32)) - 1
    ei = new_idx[ei]
    return (x, ei, ea2, batch)
